# baseline (speedup 1.0000x reference)
import math

import jax
import jax.numpy as jnp
from jax import lax
from jax.experimental import pallas as pl
from jax.experimental.pallas import tpu as pltpu
from jax.experimental.pallas import tpu_sc as plsc

N = 512
E = 8192
HID = 128
F = 128
NRBF = 64
H = 8
HD = F // H
NZ = 4
M = NZ * N
CUT_HI = 5.0

EBLK = 1024
NW = 32
EPW = E // NW
CHUNK = 128
NCH = EPW // CHUNK
RPS = M // 16


def _silu(t):
    return t * jax.nn.sigmoid(t)


def _dott(a, b):
    return lax.dot_general(a, b, (((1,), (1,)), ((), ())),
                           preferred_element_type=jnp.float32)


def _filter_body(ea_ref, ew_ref, x_ref, fn1_ref, fn1b_ref, fn2_ref, fn2b_ref,
                 lin1_ref, w_ref, h_ref):
    t = _dott(ea_ref[...], fn1_ref[...]) + fn1b_ref[...]
    t = _silu(t)
    w = _dott(t, fn2_ref[...]) + fn2b_ref[...]
    ew = ew_ref[...]
    c = 0.5 * (jnp.cos(ew * (math.pi / CUT_HI)) + 1.0)
    c = c * (ew < CUT_HI).astype(jnp.float32)
    w_ref[...] = w * c[:, None]

    @pl.when(pl.program_id(0) == 0)
    def _():
        h_ref[...] = _dott(x_ref[...], lin1_ref[...])


def _run_filter(edge_attr, edge_weight, x, fn1, fn1b, fn2, fn2b, lin1):
    grid = E // EBLK
    return pl.pallas_call(
        _filter_body,
        grid=(grid,),
        in_specs=[
            pl.BlockSpec((EBLK, NRBF), lambda e: (e, 0)),
            pl.BlockSpec((EBLK,), lambda e: (e,)),
            pl.BlockSpec((N, HID), lambda e: (0, 0)),
            pl.BlockSpec((F, NRBF), lambda e: (0, 0)),
            pl.BlockSpec((1, F), lambda e: (0, 0)),
            pl.BlockSpec((F, F), lambda e: (0, 0)),
            pl.BlockSpec((1, F), lambda e: (0, 0)),
            pl.BlockSpec((F, HID), lambda e: (0, 0)),
        ],
        out_specs=[
            pl.BlockSpec((EBLK, F), lambda e: (e, 0)),
            pl.BlockSpec((N, F), lambda e: (0, 0)),
        ],
        out_shape=[
            jax.ShapeDtypeStruct((E, F), jnp.float32),
            jax.ShapeDtypeStruct((N, F), jnp.float32),
        ],
    )(edge_attr, edge_weight, x, fn1, fn1b, fn2, fn2b, lin1)


def _sc_body(h_hbm, w_hbm, src_hbm, dst_hbm, z_hbm, yp_hbm,
             src_v, dst_v, code_v, z_a, z_b, rows_a, rows_b, wrows_a, wrows_b,
             ysh, sem_ix, sem_z, sem_h, sem_w, sem_sc):
    c = lax.axis_index("c")
    s = lax.axis_index("s")
    wid = s * 2 + c
    base = wid * EPW

    cp_src = pltpu.async_copy(src_hbm.at[pl.ds(wid * NCH, NCH)], src_v, sem_ix)
    cp_dst = pltpu.async_copy(dst_hbm.at[pl.ds(wid * NCH, NCH)], dst_v, sem_ix)

    def _zrow(r, carry):
        r4 = r * 4
        for rr in range(4):
            for k in range(F // 16):
                rows_a[r4 + rr, pl.ds(k * 16, 16)] = jnp.zeros((16,), jnp.float32)
        return carry

    lax.fori_loop(0, RPS // 4, _zrow, 0)
    pltpu.sync_copy(rows_a, ysh.at[pl.ds(s * RPS, RPS)])

    cp_src.wait()
    cp_dst.wait()

    g_a = pltpu.async_copy(z_hbm.at[src_v.at[0]], z_a, sem_z)
    g_b = pltpu.async_copy(z_hbm.at[src_v.at[1]], z_b, sem_z)
    h_a = pltpu.async_copy(h_hbm.at[src_v.at[0]], rows_a, sem_h)
    h_b = pltpu.async_copy(h_hbm.at[src_v.at[1]], rows_b, sem_h)
    w_a = pltpu.async_copy(w_hbm.at[pl.ds(base, CHUNK)], wrows_a, sem_w)
    w_b = pltpu.async_copy(w_hbm.at[pl.ds(base + CHUNK, CHUNK)], wrows_b, sem_w)

    g_a.wait()
    g_b.wait()
    for j, z_v in ((0, z_a), (1, z_b)):
        for i in range(CHUNK // 16):
            sl = pl.ds(i * 16, 16)
            code_v[j, sl] = z_v[sl] * N + dst_v[j, sl]

    plsc.subcore_barrier()

    def _mul(rows_v, wrows_v):
        def _mrow(r, carry):
            r4 = r * 4
            for rr in range(4):
                for k in range(F // 16):
                    sl = pl.ds(k * 16, 16)
                    rows_v[r4 + rr, sl] = rows_v[r4 + rr, sl] * wrows_v[r4 + rr, sl]
            return carry

        lax.fori_loop(0, CHUNK // 4, _mrow, 0)

    h_a.wait()
    h_b.wait()
    w_a.wait()
    w_b.wait()
    _mul(rows_a, wrows_a)
    sc_a = pltpu.async_copy(rows_a, ysh.at[code_v.at[0]], sem_sc, add=True)
    _mul(rows_b, wrows_b)
    sc_b = pltpu.async_copy(rows_b, ysh.at[code_v.at[1]], sem_sc, add=True)
    sc_a.wait()
    sc_b.wait()

    plsc.subcore_barrier()
    pltpu.sync_copy(ysh.at[pl.ds(s * RPS, RPS)], yp_hbm.at[c, pl.ds(s * RPS, RPS)])


def _run_sc(h, w, src2, dst2, z):
    mesh = plsc.VectorSubcoreMesh(core_axis_name="c", subcore_axis_name="s")
    return pl.kernel(
        _sc_body,
        out_type=jax.ShapeDtypeStruct((2, M, F), jnp.float32),
        mesh=mesh,
        scratch_types=[
            pltpu.VMEM((NCH, CHUNK), jnp.int32),
            pltpu.VMEM((NCH, CHUNK), jnp.int32),
            pltpu.VMEM((NCH, CHUNK), jnp.int32),
            pltpu.VMEM((CHUNK,), jnp.int32),
            pltpu.VMEM((CHUNK,), jnp.int32),
            pltpu.VMEM((CHUNK, F), jnp.float32),
            pltpu.VMEM((CHUNK, F), jnp.float32),
            pltpu.VMEM((CHUNK, F), jnp.float32),
            pltpu.VMEM((CHUNK, F), jnp.float32),
            pltpu.VMEM_SHARED((M, F), jnp.float32),
            pltpu.SemaphoreType.DMA,
            pltpu.SemaphoreType.DMA,
            pltpu.SemaphoreType.DMA,
            pltpu.SemaphoreType.DMA,
            pltpu.SemaphoreType.DMA,
        ],
    )(h, w, src2, dst2, z)


def _attn_body(yp_ref, q_w_ref, qb_ref, k_w_ref, kb_ref, v_w_ref, vb_ref,
               o1_ref, o1b_ref, o2_ref, o2b_ref, out_ref):
    y = yp_ref[0] + yp_ref[1]
    q = _dott(y, q_w_ref[...]) + qb_ref[...]
    k = _dott(y, k_w_ref[...]) + kb_ref[...]
    v = _dott(y, v_w_ref[...]) + vb_ref[...]
    ri = lax.broadcasted_iota(jnp.int32, (F, F), 0) // HD
    ci = lax.broadcasted_iota(jnp.int32, (F, F), 1) // HD
    p = (ri == ci).astype(jnp.float32)

    node = jnp.zeros((N, F), jnp.float32)
    for i in range(NZ):
        qi = q[i * N:(i + 1) * N]
        acc = jnp.zeros((N, F), jnp.float32)
        for j in range(NZ):
            kj = k[j * N:(j + 1) * N]
            t = jnp.dot(qi * kj, p, preferred_element_type=jnp.float32)
            acc = acc + _silu(t) * v[j * N:(j + 1) * N]
        node = node + _dott(acc, o1_ref[...]) + o1b_ref[...]

    o = _dott(node, o2_ref[...])
    out_ref[...] = _silu(o + o2b_ref[...])


def _run_attn(yp, q_w, qb, k_w, kb, v_w, vb, o1, o1b, o2, o2b):
    return pl.pallas_call(
        _attn_body,
        out_shape=jax.ShapeDtypeStruct((N, F), jnp.float32),
    )(yp, q_w, qb, k_w, kb, v_w, vb, o1, o1b, o2, o2b)


def kernel(x, z, edge_index, edge_weight, edge_attr, lin1_w, fn1_w, fn1_b,
           fn2_w, fn2_b, q_w, q_b, k_w, k_b, v_w, v_b, o1_w, o1_b, o2_w, o2_b):
    src2 = edge_index[0].astype(jnp.int32).reshape(E // CHUNK, CHUNK)
    dst2 = edge_index[1].astype(jnp.int32).reshape(E // CHUNK, CHUNK)
    zi = z.astype(jnp.int32)

    w, h = _run_filter(edge_attr, edge_weight, x, fn1_w, fn1_b.reshape(1, F),
                       fn2_w, fn2_b.reshape(1, F), lin1_w)
    yp = _run_sc(h, w, src2, dst2, zi)
    return _run_attn(yp, q_w, q_b.reshape(1, F), k_w, k_b.reshape(1, F),
                     v_w, v_b.reshape(1, F), o1_w, o1_b.reshape(1, F),
                     o2_w, o2_b.reshape(1, F))


# speedup vs baseline: 1.0614x; 1.0614x over previous
import math

import jax
import jax.numpy as jnp
from jax import lax
from jax.experimental import pallas as pl
from jax.experimental.pallas import tpu as pltpu
from jax.experimental.pallas import tpu_sc as plsc

N = 512
E = 8192
HID = 128
F = 128
NRBF = 64
H = 8
HD = F // H
NZ = 4
M = NZ * N
CUT_HI = 5.0

EBLK = 4096
NW = 32
EPW = E // NW
CHUNK = 128
NCH = EPW // CHUNK
RPS = M // 16


def _silu(t):
    return t * jax.nn.sigmoid(t)


def _dott(a, b):
    return lax.dot_general(a, b, (((1,), (1,)), ((), ())),
                           preferred_element_type=jnp.float32)


def _filter_body(ea_ref, ew_ref, x_ref, fn1_ref, fn1b_ref, fn2_ref, fn2b_ref,
                 lin1_ref, w_ref, h_ref):
    t = _dott(ea_ref[...], fn1_ref[...]) + fn1b_ref[...]
    t = _silu(t)
    w = _dott(t, fn2_ref[...]) + fn2b_ref[...]
    ew = ew_ref[...]
    c = 0.5 * (jnp.cos(ew * (math.pi / CUT_HI)) + 1.0)
    c = c * (ew < CUT_HI).astype(jnp.float32)
    w_ref[...] = w * c[:, None]

    @pl.when(pl.program_id(0) == 0)
    def _():
        h_ref[...] = _dott(x_ref[...], lin1_ref[...])


def _run_filter(edge_attr, edge_weight, x, fn1, fn1b, fn2, fn2b, lin1):
    grid = E // EBLK
    return pl.pallas_call(
        _filter_body,
        grid=(grid,),
        in_specs=[
            pl.BlockSpec((EBLK, NRBF), lambda e: (e, 0)),
            pl.BlockSpec((EBLK,), lambda e: (e,)),
            pl.BlockSpec((N, HID), lambda e: (0, 0)),
            pl.BlockSpec((F, NRBF), lambda e: (0, 0)),
            pl.BlockSpec((1, F), lambda e: (0, 0)),
            pl.BlockSpec((F, F), lambda e: (0, 0)),
            pl.BlockSpec((1, F), lambda e: (0, 0)),
            pl.BlockSpec((F, HID), lambda e: (0, 0)),
        ],
        out_specs=[
            pl.BlockSpec((EBLK, F), lambda e: (e, 0)),
            pl.BlockSpec((N, F), lambda e: (0, 0)),
        ],
        out_shape=[
            jax.ShapeDtypeStruct((E, F), jnp.float32),
            jax.ShapeDtypeStruct((N, F), jnp.float32),
        ],
    )(edge_attr, edge_weight, x, fn1, fn1b, fn2, fn2b, lin1)


def _sc_body(h_hbm, w_hbm, ei_hbm, z_hbm, yp_hbm,
             src_v, dst_v, code_v, z_a, z_b, rows_a, rows_b, wrows_a, wrows_b,
             ysh, sem_ix, sem_z, sem_h, sem_w, sem_sc):
    c = lax.axis_index("c")
    s = lax.axis_index("s")
    wid = s * 2 + c
    base = wid * EPW

    cp_src = pltpu.async_copy(ei_hbm.at[0, pl.ds(wid * NCH, NCH)], src_v, sem_ix)
    cp_dst = pltpu.async_copy(ei_hbm.at[1, pl.ds(wid * NCH, NCH)], dst_v, sem_ix)

    with jax.named_scope("zero"):
        def _zrow(r, carry):
            r4 = r * 4
            for rr in range(4):
                for k in range(F // 16):
                    rows_a[r4 + rr, pl.ds(k * 16, 16)] = jnp.zeros((16,), jnp.float32)
            return carry

        lax.fori_loop(0, RPS // 4, _zrow, 0)
        pltpu.sync_copy(rows_a, ysh.at[pl.ds(s * RPS, RPS)])

    cp_src.wait()
    cp_dst.wait()

    g_a = pltpu.async_copy(z_hbm.at[src_v.at[0]], z_a, sem_z)
    g_b = pltpu.async_copy(z_hbm.at[src_v.at[1]], z_b, sem_z)
    h_a = pltpu.async_copy(h_hbm.at[src_v.at[0]], rows_a, sem_h)
    h_b = pltpu.async_copy(h_hbm.at[src_v.at[1]], rows_b, sem_h)
    w_a = pltpu.async_copy(w_hbm.at[pl.ds(base, CHUNK)], wrows_a, sem_w)
    w_b = pltpu.async_copy(w_hbm.at[pl.ds(base + CHUNK, CHUNK)], wrows_b, sem_w)

    with jax.named_scope("codes"):
        g_a.wait()
        g_b.wait()
        for j, z_v in ((0, z_a), (1, z_b)):
            for i in range(CHUNK // 16):
                sl = pl.ds(i * 16, 16)
                code_v[j, sl] = z_v[sl] * N + dst_v[j, sl]

    with jax.named_scope("barrier1"):
        plsc.subcore_barrier()

    def _mul(rows_v, wrows_v):
        def _mrow(r, carry):
            r4 = r * 4
            for rr in range(4):
                for k in range(F // 16):
                    sl = pl.ds(k * 16, 16)
                    rows_v[r4 + rr, sl] = rows_v[r4 + rr, sl] * wrows_v[r4 + rr, sl]
            return carry

        lax.fori_loop(0, CHUNK // 4, _mrow, 0)

    with jax.named_scope("indma"):
        h_a.wait()
        h_b.wait()
        w_a.wait()
        w_b.wait()
    with jax.named_scope("mulscat"):
        _mul(rows_a, wrows_a)
        sc_a = pltpu.async_copy(rows_a, ysh.at[code_v.at[0]], sem_sc, add=True)
        _mul(rows_b, wrows_b)
        sc_b = pltpu.async_copy(rows_b, ysh.at[code_v.at[1]], sem_sc, add=True)
        sc_a.wait()
        sc_b.wait()

    with jax.named_scope("barrier2"):
        plsc.subcore_barrier()
    with jax.named_scope("writeback"):
        pltpu.sync_copy(ysh.at[pl.ds(s * RPS, RPS)], yp_hbm.at[c, pl.ds(s * RPS, RPS)])


def _run_sc(h, w, ei3, z):
    mesh = plsc.VectorSubcoreMesh(core_axis_name="c", subcore_axis_name="s")
    return pl.kernel(
        _sc_body,
        out_type=jax.ShapeDtypeStruct((2, M, F), jnp.float32),
        mesh=mesh,
        scratch_types=[
            pltpu.VMEM((NCH, CHUNK), jnp.int32),
            pltpu.VMEM((NCH, CHUNK), jnp.int32),
            pltpu.VMEM((NCH, CHUNK), jnp.int32),
            pltpu.VMEM((CHUNK,), jnp.int32),
            pltpu.VMEM((CHUNK,), jnp.int32),
            pltpu.VMEM((CHUNK, F), jnp.float32),
            pltpu.VMEM((CHUNK, F), jnp.float32),
            pltpu.VMEM((CHUNK, F), jnp.float32),
            pltpu.VMEM((CHUNK, F), jnp.float32),
            pltpu.VMEM_SHARED((M, F), jnp.float32),
            pltpu.SemaphoreType.DMA,
            pltpu.SemaphoreType.DMA,
            pltpu.SemaphoreType.DMA,
            pltpu.SemaphoreType.DMA,
            pltpu.SemaphoreType.DMA,
        ],
    )(h, w, ei3, z)


def _attn_body(yp_ref, q_w_ref, qb_ref, k_w_ref, kb_ref, v_w_ref, vb_ref,
               o1_ref, o1b_ref, o2_ref, o2b_ref, out_ref):
    y = yp_ref[0] + yp_ref[1]
    q = _dott(y, q_w_ref[...]) + qb_ref[...]
    k = _dott(y, k_w_ref[...]) + kb_ref[...]
    v = _dott(y, v_w_ref[...]) + vb_ref[...]
    ri = lax.broadcasted_iota(jnp.int32, (F, F), 0) // HD
    ci = lax.broadcasted_iota(jnp.int32, (F, F), 1) // HD
    p = (ri == ci).astype(jnp.float32)

    node = jnp.zeros((N, F), jnp.float32)
    for i in range(NZ):
        qi = q[i * N:(i + 1) * N]
        acc = jnp.zeros((N, F), jnp.float32)
        for j in range(NZ):
            kj = k[j * N:(j + 1) * N]
            t = jnp.dot(qi * kj, p, preferred_element_type=jnp.float32)
            acc = acc + _silu(t) * v[j * N:(j + 1) * N]
        node = node + _dott(acc, o1_ref[...]) + o1b_ref[...]

    o = _dott(node, o2_ref[...])
    out_ref[...] = _silu(o + o2b_ref[...])


def _run_attn(yp, q_w, qb, k_w, kb, v_w, vb, o1, o1b, o2, o2b):
    return pl.pallas_call(
        _attn_body,
        out_shape=jax.ShapeDtypeStruct((N, F), jnp.float32),
    )(yp, q_w, qb, k_w, kb, v_w, vb, o1, o1b, o2, o2b)


def kernel(x, z, edge_index, edge_weight, edge_attr, lin1_w, fn1_w, fn1_b,
           fn2_w, fn2_b, q_w, q_b, k_w, k_b, v_w, v_b, o1_w, o1_b, o2_w, o2_b):
    ei3 = edge_index.astype(jnp.int32).reshape(2, E // CHUNK, CHUNK)
    zi = z.astype(jnp.int32)

    w, h = _run_filter(edge_attr, edge_weight, x, fn1_w, fn1_b.reshape(1, F),
                       fn2_w, fn2_b.reshape(1, F), lin1_w)
    yp = _run_sc(h, w, ei3, zi)
    return _run_attn(yp, q_w, q_b.reshape(1, F), k_w, k_b.reshape(1, F),
                     v_w, v_b.reshape(1, F), o1_w, o1_b.reshape(1, F),
                     o2_w, o2_b.reshape(1, F))


# speedup vs baseline: 1.1242x; 1.0592x over previous
import math

import jax
import jax.numpy as jnp
from jax import lax
from jax.experimental import pallas as pl
from jax.experimental.pallas import tpu as pltpu
from jax.experimental.pallas import tpu_sc as plsc

N = 512
E = 8192
HID = 128
F = 128
NRBF = 64
H = 8
HD = F // H
NZ = 4
M = NZ * N
CUT_HI = 5.0

EBLK = 4096
NW = 32
EPW = E // NW
CHUNK = 128
NCH = EPW // CHUNK
RPS = M // 16


def _silu(t):
    return t * jax.nn.sigmoid(t)


def _dott(a, b):
    return lax.dot_general(a, b, (((1,), (1,)), ((), ())),
                           preferred_element_type=jnp.float32)


def _filter_body(ea_ref, ew_ref, src_ref, dst_ref, zf_ref, x_ref, fn1_ref,
                 fn1b_ref, fn2_ref, fn2b_ref, lin1_ref, w_ref, code_ref, h_ref):
    t = _dott(ea_ref[...], fn1_ref[...]) + fn1b_ref[...]
    t = _silu(t)
    w = _dott(t, fn2_ref[...]) + fn2b_ref[...]
    ew = ew_ref[...]
    c = 0.5 * (jnp.cos(ew * (math.pi / CUT_HI)) + 1.0)
    c = c * (ew < CUT_HI).astype(jnp.float32)
    w_ref[...] = w * c[:, None]

    src = src_ref[...]
    oh = (src[:, None] == lax.broadcasted_iota(jnp.int32, (EBLK, N), 1))
    zsrc = jnp.dot(oh.astype(jnp.float32), zf_ref[...][:, None],
                   preferred_element_type=jnp.float32)
    code = zsrc[:, 0].astype(jnp.int32) * N + dst_ref[...]
    code_ref[...] = code.reshape(EBLK // CHUNK, CHUNK)

    @pl.when(pl.program_id(0) == 0)
    def _():
        h_ref[...] = _dott(x_ref[...], lin1_ref[...])


def _run_filter(edge_attr, edge_weight, src, dst, zf, x, fn1, fn1b, fn2, fn2b,
                lin1):
    grid = E // EBLK
    return pl.pallas_call(
        _filter_body,
        grid=(grid,),
        in_specs=[
            pl.BlockSpec((EBLK, NRBF), lambda e: (e, 0)),
            pl.BlockSpec((EBLK,), lambda e: (e,)),
            pl.BlockSpec((EBLK,), lambda e: (e,)),
            pl.BlockSpec((EBLK,), lambda e: (e,)),
            pl.BlockSpec((N,), lambda e: (0,)),
            pl.BlockSpec((N, HID), lambda e: (0, 0)),
            pl.BlockSpec((F, NRBF), lambda e: (0, 0)),
            pl.BlockSpec((1, F), lambda e: (0, 0)),
            pl.BlockSpec((F, F), lambda e: (0, 0)),
            pl.BlockSpec((1, F), lambda e: (0, 0)),
            pl.BlockSpec((F, HID), lambda e: (0, 0)),
        ],
        out_specs=[
            pl.BlockSpec((EBLK, F), lambda e: (e, 0)),
            pl.BlockSpec((EBLK // CHUNK, CHUNK), lambda e: (e, 0)),
            pl.BlockSpec((N, F), lambda e: (0, 0)),
        ],
        out_shape=[
            jax.ShapeDtypeStruct((E, F), jnp.float32),
            jax.ShapeDtypeStruct((E // CHUNK, CHUNK), jnp.int32),
            jax.ShapeDtypeStruct((N, F), jnp.float32),
        ],
    )(edge_attr, edge_weight, src, dst, zf, x, fn1, fn1b, fn2, fn2b, lin1)


def _sc_body(h_hbm, w_hbm, ei_hbm, code_hbm, yp_hbm,
             src_v, code_v, rows_a, rows_b, wrows_a, wrows_b,
             ysh, sem_ix, sem_h, sem_w, sem_sc):
    c = lax.axis_index("c")
    s = lax.axis_index("s")
    wid = s * 2 + c
    base = wid * EPW

    w_a = pltpu.async_copy(w_hbm.at[pl.ds(base, CHUNK)], wrows_a, sem_w)
    w_b = pltpu.async_copy(w_hbm.at[pl.ds(base + CHUNK, CHUNK)], wrows_b, sem_w)

    cp_src = pltpu.async_copy(ei_hbm.at[0, pl.ds(base, EPW)], src_v, sem_ix)
    cp_code = pltpu.async_copy(code_hbm.at[pl.ds(wid * NCH, NCH)], code_v, sem_ix)

    with jax.named_scope("zero"):
        def _zrow(r, carry):
            r4 = r * 4
            for rr in range(4):
                for k in range(F // 16):
                    rows_a[r4 + rr, pl.ds(k * 16, 16)] = jnp.zeros((16,), jnp.float32)
            return carry

        lax.fori_loop(0, RPS // 4, _zrow, 0)
        pltpu.sync_copy(rows_a, ysh.at[pl.ds(s * RPS, RPS)])

    with jax.named_scope("ixwait"):
        cp_src.wait()
        cp_code.wait()

    h_a = pltpu.async_copy(h_hbm.at[src_v.at[pl.ds(0, CHUNK)]], rows_a, sem_h)
    h_b = pltpu.async_copy(h_hbm.at[src_v.at[pl.ds(CHUNK, CHUNK)]], rows_b, sem_h)

    with jax.named_scope("barrier1"):
        plsc.subcore_barrier()

    with jax.named_scope("indma"):
        h_a.wait()
        h_b.wait()
        w_a.wait()
        w_b.wait()

    def _mul(rows_v, wrows_v):
        def _mrow(r, carry):
            r4 = r * 4
            for rr in range(4):
                for k in range(F // 16):
                    sl = pl.ds(k * 16, 16)
                    rows_v[r4 + rr, sl] = rows_v[r4 + rr, sl] * wrows_v[r4 + rr, sl]
            return carry

        lax.fori_loop(0, CHUNK // 4, _mrow, 0)

    with jax.named_scope("mulscat"):
        _mul(rows_a, wrows_a)
        sc_a = pltpu.async_copy(rows_a, ysh.at[code_v.at[0]], sem_sc, add=True)
        _mul(rows_b, wrows_b)
        sc_b = pltpu.async_copy(rows_b, ysh.at[code_v.at[1]], sem_sc, add=True)
        sc_a.wait()
        sc_b.wait()

    with jax.named_scope("barrier2"):
        plsc.subcore_barrier()
    with jax.named_scope("writeback"):
        pltpu.sync_copy(ysh.at[pl.ds(s * RPS, RPS)], yp_hbm.at[c, pl.ds(s * RPS, RPS)])


def _run_sc(h, w, ei, codes):
    mesh = plsc.VectorSubcoreMesh(core_axis_name="c", subcore_axis_name="s")
    return pl.kernel(
        _sc_body,
        out_type=jax.ShapeDtypeStruct((2, M, F), jnp.float32),
        mesh=mesh,
        scratch_types=[
            pltpu.VMEM((EPW,), jnp.int32),
            pltpu.VMEM((NCH, CHUNK), jnp.int32),
            pltpu.VMEM((CHUNK, F), jnp.float32),
            pltpu.VMEM((CHUNK, F), jnp.float32),
            pltpu.VMEM((CHUNK, F), jnp.float32),
            pltpu.VMEM((CHUNK, F), jnp.float32),
            pltpu.VMEM_SHARED((M, F), jnp.float32),
            pltpu.SemaphoreType.DMA,
            pltpu.SemaphoreType.DMA,
            pltpu.SemaphoreType.DMA,
            pltpu.SemaphoreType.DMA,
        ],
    )(h, w, ei, codes)


def _attn_body(yp_ref, q_w_ref, qb_ref, k_w_ref, kb_ref, v_w_ref, vb_ref,
               o1_ref, o1b_ref, o2_ref, o2b_ref, out_ref):
    y = yp_ref[0] + yp_ref[1]
    q = _dott(y, q_w_ref[...]) + qb_ref[...]
    k = _dott(y, k_w_ref[...]) + kb_ref[...]
    v = _dott(y, v_w_ref[...]) + vb_ref[...]
    ri = lax.broadcasted_iota(jnp.int32, (F, F), 0) // HD
    ci = lax.broadcasted_iota(jnp.int32, (F, F), 1) // HD
    p = (ri == ci).astype(jnp.float32)

    node = jnp.zeros((N, F), jnp.float32)
    for i in range(NZ):
        qi = q[i * N:(i + 1) * N]
        acc = jnp.zeros((N, F), jnp.float32)
        for j in range(NZ):
            kj = k[j * N:(j + 1) * N]
            t = jnp.dot(qi * kj, p, preferred_element_type=jnp.float32)
            acc = acc + _silu(t) * v[j * N:(j + 1) * N]
        node = node + _dott(acc, o1_ref[...]) + o1b_ref[...]

    o = _dott(node, o2_ref[...])
    out_ref[...] = _silu(o + o2b_ref[...])


def _run_attn(yp, q_w, qb, k_w, kb, v_w, vb, o1, o1b, o2, o2b):
    return pl.pallas_call(
        _attn_body,
        out_shape=jax.ShapeDtypeStruct((N, F), jnp.float32),
    )(yp, q_w, qb, k_w, kb, v_w, vb, o1, o1b, o2, o2b)


def kernel(x, z, edge_index, edge_weight, edge_attr, lin1_w, fn1_w, fn1_b,
           fn2_w, fn2_b, q_w, q_b, k_w, k_b, v_w, v_b, o1_w, o1_b, o2_w, o2_b):
    ei = edge_index.astype(jnp.int32)
    zf = z.astype(jnp.float32)

    w, codes, h = _run_filter(edge_attr, edge_weight, ei[0], ei[1], zf, x,
                              fn1_w, fn1_b.reshape(1, F), fn2_w,
                              fn2_b.reshape(1, F), lin1_w)
    yp = _run_sc(h, w, ei, codes)
    return _run_attn(yp, q_w, q_b.reshape(1, F), k_w, k_b.reshape(1, F),
                     v_w, v_b.reshape(1, F), o1_w, o1_b.reshape(1, F),
                     o2_w, o2_b.reshape(1, F))


# speedup vs baseline: 1.1256x; 1.0012x over previous
import math

import jax
import jax.numpy as jnp
from jax import lax
from jax.experimental import pallas as pl
from jax.experimental.pallas import tpu as pltpu
from jax.experimental.pallas import tpu_sc as plsc

N = 512
E = 8192
HID = 128
F = 128
NRBF = 64
H = 8
HD = F // H
NZ = 4
M = NZ * N
CUT_HI = 5.0

EBLK = 4096
NW = 32
EPW = E // NW
CHUNK = 128
NCH = EPW // CHUNK
RPS = M // 16


def _silu(t):
    return t * jax.nn.sigmoid(t)


def _dott(a, b):
    return lax.dot_general(a, b, (((1,), (1,)), ((), ())),
                           preferred_element_type=jnp.float32)


def _filter_body(ea_ref, ew_ref, src_ref, dst_ref, zf_ref, x_ref, fn1_ref,
                 fn1b_ref, fn2_ref, fn2b_ref, lin1_ref, w_ref, code_ref, h_ref):
    t = _dott(ea_ref[...], fn1_ref[...]) + fn1b_ref[...]
    t = _silu(t)
    w = _dott(t, fn2_ref[...]) + fn2b_ref[...]
    ew = ew_ref[...]
    c = 0.5 * (jnp.cos(ew * (math.pi / CUT_HI)) + 1.0)
    c = c * (ew < CUT_HI).astype(jnp.float32)
    w_ref[...] = w * c[:, None]

    src = src_ref[...]
    oh = (src[:, None] == lax.broadcasted_iota(jnp.int32, (EBLK, N), 1))
    zsrc = jnp.dot(oh.astype(jnp.float32), zf_ref[...][:, None],
                   preferred_element_type=jnp.float32)
    code = zsrc[:, 0].astype(jnp.int32) * N + dst_ref[...]
    code_ref[...] = code.reshape(EBLK // CHUNK, CHUNK)

    @pl.when(pl.program_id(0) == 0)
    def _():
        h_ref[...] = _dott(x_ref[...], lin1_ref[...])


def _run_filter(edge_attr, edge_weight, src, dst, zf, x, fn1, fn1b, fn2, fn2b,
                lin1):
    grid = E // EBLK
    return pl.pallas_call(
        _filter_body,
        grid=(grid,),
        in_specs=[
            pl.BlockSpec((EBLK, NRBF), lambda e: (e, 0)),
            pl.BlockSpec((EBLK,), lambda e: (e,)),
            pl.BlockSpec((EBLK,), lambda e: (e,)),
            pl.BlockSpec((EBLK,), lambda e: (e,)),
            pl.BlockSpec((N,), lambda e: (0,)),
            pl.BlockSpec((N, HID), lambda e: (0, 0)),
            pl.BlockSpec((F, NRBF), lambda e: (0, 0)),
            pl.BlockSpec((1, F), lambda e: (0, 0)),
            pl.BlockSpec((F, F), lambda e: (0, 0)),
            pl.BlockSpec((1, F), lambda e: (0, 0)),
            pl.BlockSpec((F, HID), lambda e: (0, 0)),
        ],
        out_specs=[
            pl.BlockSpec((EBLK, F), lambda e: (e, 0)),
            pl.BlockSpec((EBLK // CHUNK, CHUNK), lambda e: (e, 0)),
            pl.BlockSpec((N, F), lambda e: (0, 0)),
        ],
        out_shape=[
            jax.ShapeDtypeStruct((E, F), jnp.float32),
            jax.ShapeDtypeStruct((E // CHUNK, CHUNK), jnp.int32),
            jax.ShapeDtypeStruct((N, F), jnp.float32),
        ],
    )(edge_attr, edge_weight, src, dst, zf, x, fn1, fn1b, fn2, fn2b, lin1)


def _sc_body(h_hbm, w_hbm, src_hbm, code_hbm, yp_hbm,
             src_v, code_v, rows_a, rows_b, wrows_a, wrows_b,
             ysh, sem_ix, sem_h, sem_w, sem_sc):
    c = lax.axis_index("c")
    s = lax.axis_index("s")
    wid = s * 2 + c
    base = wid * EPW

    w_a = pltpu.async_copy(w_hbm.at[pl.ds(base, CHUNK)], wrows_a, sem_w)
    w_b = pltpu.async_copy(w_hbm.at[pl.ds(base + CHUNK, CHUNK)], wrows_b, sem_w)

    cp_sa = pltpu.async_copy(src_hbm.at[pl.ds(base, CHUNK)], src_v.at[0], sem_ix)
    cp_sb = pltpu.async_copy(src_hbm.at[pl.ds(base + CHUNK, CHUNK)], src_v.at[1], sem_ix)
    cp_code = pltpu.async_copy(code_hbm.at[pl.ds(wid * NCH, NCH)], code_v, sem_ix)

    with jax.named_scope("zero"):
        def _zrow(r, carry):
            r4 = r * 4
            for rr in range(4):
                for k in range(F // 16):
                    rows_a[r4 + rr, pl.ds(k * 16, 16)] = jnp.zeros((16,), jnp.float32)
            return carry

        lax.fori_loop(0, RPS // 4, _zrow, 0)
        pltpu.sync_copy(rows_a, ysh.at[pl.ds(s * RPS, RPS)])

    with jax.named_scope("ixwait"):
        cp_sa.wait()
        cp_sb.wait()
        cp_code.wait()

    h_a = pltpu.async_copy(h_hbm.at[src_v.at[0]], rows_a, sem_h)
    h_b = pltpu.async_copy(h_hbm.at[src_v.at[1]], rows_b, sem_h)

    with jax.named_scope("barrier1"):
        plsc.subcore_barrier()

    with jax.named_scope("indma"):
        h_a.wait()
        h_b.wait()
        w_a.wait()
        w_b.wait()

    def _mul(rows_v, wrows_v):
        def _mrow(r, carry):
            r4 = r * 4
            for rr in range(4):
                for k in range(F // 16):
                    sl = pl.ds(k * 16, 16)
                    rows_v[r4 + rr, sl] = rows_v[r4 + rr, sl] * wrows_v[r4 + rr, sl]
            return carry

        lax.fori_loop(0, CHUNK // 4, _mrow, 0)

    with jax.named_scope("mulscat"):
        _mul(rows_a, wrows_a)
        sc_a = pltpu.async_copy(rows_a, ysh.at[code_v.at[0]], sem_sc, add=True)
        _mul(rows_b, wrows_b)
        sc_b = pltpu.async_copy(rows_b, ysh.at[code_v.at[1]], sem_sc, add=True)
        sc_a.wait()
        sc_b.wait()

    with jax.named_scope("barrier2"):
        plsc.subcore_barrier()
    with jax.named_scope("writeback"):
        pltpu.sync_copy(ysh.at[pl.ds(s * RPS, RPS)], yp_hbm.at[c, pl.ds(s * RPS, RPS)])


def _run_sc(h, w, src, codes):
    mesh = plsc.VectorSubcoreMesh(core_axis_name="c", subcore_axis_name="s")
    return pl.kernel(
        _sc_body,
        out_type=jax.ShapeDtypeStruct((2, M, F), jnp.float32),
        mesh=mesh,
        scratch_types=[
            pltpu.VMEM((NCH, CHUNK), jnp.int32),
            pltpu.VMEM((NCH, CHUNK), jnp.int32),
            pltpu.VMEM((CHUNK, F), jnp.float32),
            pltpu.VMEM((CHUNK, F), jnp.float32),
            pltpu.VMEM((CHUNK, F), jnp.float32),
            pltpu.VMEM((CHUNK, F), jnp.float32),
            pltpu.VMEM_SHARED((M, F), jnp.float32),
            pltpu.SemaphoreType.DMA,
            pltpu.SemaphoreType.DMA,
            pltpu.SemaphoreType.DMA,
            pltpu.SemaphoreType.DMA,
        ],
    )(h, w, src, codes)


def _attn_body(yp_ref, q_w_ref, qb_ref, k_w_ref, kb_ref, v_w_ref, vb_ref,
               o1_ref, o1b_ref, o2_ref, o2b_ref, out_ref):
    y = yp_ref[0] + yp_ref[1]
    q = _dott(y, q_w_ref[...]) + qb_ref[...]
    k = _dott(y, k_w_ref[...]) + kb_ref[...]
    v = _dott(y, v_w_ref[...]) + vb_ref[...]
    ri = lax.broadcasted_iota(jnp.int32, (F, F), 0) // HD
    ci = lax.broadcasted_iota(jnp.int32, (F, F), 1) // HD
    p = (ri == ci).astype(jnp.float32)

    node = jnp.zeros((N, F), jnp.float32)
    for i in range(NZ):
        qi = q[i * N:(i + 1) * N]
        acc = jnp.zeros((N, F), jnp.float32)
        for j in range(NZ):
            kj = k[j * N:(j + 1) * N]
            t = jnp.dot(qi * kj, p, preferred_element_type=jnp.float32)
            acc = acc + _silu(t) * v[j * N:(j + 1) * N]
        node = node + _dott(acc, o1_ref[...]) + o1b_ref[...]

    o = _dott(node, o2_ref[...])
    out_ref[...] = _silu(o + o2b_ref[...])


def _run_attn(yp, q_w, qb, k_w, kb, v_w, vb, o1, o1b, o2, o2b):
    return pl.pallas_call(
        _attn_body,
        out_shape=jax.ShapeDtypeStruct((N, F), jnp.float32),
    )(yp, q_w, qb, k_w, kb, v_w, vb, o1, o1b, o2, o2b)


def kernel(x, z, edge_index, edge_weight, edge_attr, lin1_w, fn1_w, fn1_b,
           fn2_w, fn2_b, q_w, q_b, k_w, k_b, v_w, v_b, o1_w, o1_b, o2_w, o2_b):
    ei = edge_index.astype(jnp.int32)
    zf = z.astype(jnp.float32)

    w, codes, h = _run_filter(edge_attr, edge_weight, ei[0], ei[1], zf, x,
                              fn1_w, fn1_b.reshape(1, F), fn2_w,
                              fn2_b.reshape(1, F), lin1_w)
    yp = _run_sc(h, w, ei[0], codes)
    return _run_attn(yp, q_w, q_b.reshape(1, F), k_w, k_b.reshape(1, F),
                     v_w, v_b.reshape(1, F), o1_w, o1_b.reshape(1, F),
                     o2_w, o2_b.reshape(1, F))
